# baseline (speedup 1.0000x reference)
import jax
import jax.numpy as jnp
from jax.experimental import pallas as pl
from jax.experimental.pallas import tpu as pltpu

N = 2048
E = 4
BI = 256
BIA = 256
BJT = 8
BJ = BJT * 128
LN = 128
JT = N // LN


def _pass_h0(adj_ref, x0_ref, h0_ref):
    h0_ref[...] = jnp.dot(adj_ref[...], x0_ref[...],
                          preferred_element_type=jnp.float32)


SR = 8


def _pass_a(adj_ref, e_ref, h0_ref, x0i_ref, Wee0_ref, be0_ref, Wee1_ref,
            be1_ref, T_ref, Wn0_ref, We0_ref, bn0_ref,
            x1_ref, a1_ref, a2_ref, acc0, acc1, acc2):
    j = pl.program_id(1)
    nj = pl.num_programs(1)

    w0 = [[Wee0_ref[d, dp] for dp in range(E)] for d in range(E)]
    w1 = [[Wee1_ref[d, dp] for dp in range(E)] for d in range(E)]
    b0 = [be0_ref[d] for d in range(E)]
    b1 = [be1_ref[d] for d in range(E)]

    @pl.when(j == 0)
    def _init():
        acc0[...] = jnp.zeros_like(acc0)
        acc1[...] = jnp.zeros_like(acc1)
        acc2[...] = jnp.zeros_like(acc2)

    for r in range(0, BIA, SR):
        a = adj_ref[r:r + SR]
        p = [e_ref[d, r:r + SR] for d in range(E)]
        e1 = [jnp.maximum((p[0] * w0[0][dp] + p[1] * w0[1][dp])
                          + (p[2] * w0[2][dp] + p[3] * w0[3][dp]) + b0[dp],
                          0.0)
              for dp in range(E)]
        e2 = [jnp.maximum((e1[0] * w1[0][dp] + e1[1] * w1[1][dp])
                          + (e1[2] * w1[2][dp] + e1[3] * w1[3][dp]) + b1[dp],
                          0.0)
              for dp in range(E)]
        for acc, planes in ((acc0, p), (acc1, e1), (acc2, e2)):
            for d in range(E):
                q = a * planes[d]
                f = jnp.sum(q, axis=1)
                acc[d, r:r + SR, :] += f

    @pl.when(j == nj - 1)
    def _fin():
        T = T_ref[...]

        def fold(acc):
            s = jnp.concatenate([acc[d] for d in range(E)], axis=1)
            return jnp.dot(s, T, preferred_element_type=jnp.float32)

        ea0 = fold(acc0)
        h = x0i_ref[...] + h0_ref[...]
        x1 = jnp.dot(h, Wn0_ref[...], preferred_element_type=jnp.float32)
        x1 = x1 + jnp.dot(ea0, We0_ref[...], preferred_element_type=jnp.float32)
        x1_ref[...] = jnp.maximum(x1 + bn0_ref[...], 0.0)
        a1_ref[...] = fold(acc1)
        a2_ref[...] = fold(acc2)


def _pass_b(adj_ref, x1j_ref, x1i_ref, a1_ref, a2_ref, Wn1_ref, We1_ref,
            bn1_ref, Wr_ref, Wre_ref, br_ref, out_ref, h1acc):
    j = pl.program_id(1)
    nj = pl.num_programs(1)

    @pl.when(j == 0)
    def _init():
        h1acc[...] = jnp.zeros_like(h1acc)

    h1acc[...] += jnp.dot(adj_ref[...], x1j_ref[...],
                          preferred_element_type=jnp.float32)

    @pl.when(j == nj - 1)
    def _fin():
        h = x1i_ref[...] + h1acc[...]
        x2 = jnp.dot(h, Wn1_ref[...], preferred_element_type=jnp.float32)
        x2 = x2 + jnp.dot(a1_ref[...], We1_ref[...],
                          preferred_element_type=jnp.float32)
        x2 = jnp.maximum(x2 + bn1_ref[...], 0.0)
        out = jnp.dot(x2, Wr_ref[...], preferred_element_type=jnp.float32)
        out = out + jnp.dot(a2_ref[...], Wre_ref[...],
                            preferred_element_type=jnp.float32)
        out_ref[...] = out + br_ref[...]


def kernel(node_features, edge_features, adj, Wn0, We0, bn0, Wee0, be0,
           Wn1, We1, bn1, Wee1, be1, Wr, Wre, br):
    f32 = jnp.float32
    eP = jnp.transpose(edge_features, (2, 0, 1)).reshape(E, N, JT, LN)
    adjV = adj.reshape(N, JT, LN)

    T = jnp.kron(jnp.eye(E, dtype=f32), jnp.ones((LN, 1), dtype=f32))
    bn0r = bn0.reshape(1, -1)
    bn1r = bn1.reshape(1, -1)
    brr = br.reshape(1, -1)

    gi = N // BI
    h0 = pl.pallas_call(
        _pass_h0,
        grid=(gi,),
        in_specs=[
            pl.BlockSpec((BI, N), lambda i: (i, 0)),
            pl.BlockSpec((N, 128), lambda i: (0, 0)),
        ],
        out_specs=pl.BlockSpec((BI, 128), lambda i: (i, 0)),
        out_shape=jax.ShapeDtypeStruct((N, 128), f32),
    )(adj, node_features)

    gj = N // BJ
    gia = N // BIA
    x1, a1, a2 = pl.pallas_call(
        _pass_a,
        grid=(gia, gj),
        in_specs=[
            pl.BlockSpec((BIA, BJT, LN), lambda i, j: (i, j, 0)),
            pl.BlockSpec((E, BIA, BJT, LN), lambda i, j: (0, i, j, 0)),
            pl.BlockSpec((BIA, 128), lambda i, j: (i, 0)),
            pl.BlockSpec((BIA, 128), lambda i, j: (i, 0)),
            pl.BlockSpec(memory_space=pltpu.SMEM),
            pl.BlockSpec(memory_space=pltpu.SMEM),
            pl.BlockSpec(memory_space=pltpu.SMEM),
            pl.BlockSpec(memory_space=pltpu.SMEM),
            pl.BlockSpec((E * LN, E), lambda i, j: (0, 0)),
            pl.BlockSpec((128, 256), lambda i, j: (0, 0)),
            pl.BlockSpec((E, 256), lambda i, j: (0, 0)),
            pl.BlockSpec((1, 256), lambda i, j: (0, 0)),
        ],
        out_specs=[
            pl.BlockSpec((BIA, 256), lambda i, j: (i, 0)),
            pl.BlockSpec((BIA, E), lambda i, j: (i, 0)),
            pl.BlockSpec((BIA, E), lambda i, j: (i, 0)),
        ],
        out_shape=[
            jax.ShapeDtypeStruct((N, 256), f32),
            jax.ShapeDtypeStruct((N, E), f32),
            jax.ShapeDtypeStruct((N, E), f32),
        ],
        scratch_shapes=[
            pltpu.VMEM((E, BIA, LN), f32),
            pltpu.VMEM((E, BIA, LN), f32),
            pltpu.VMEM((E, BIA, LN), f32),
        ],
    )(adjV, eP, h0, node_features, Wee0, be0, Wee1, be1, T, Wn0, We0, bn0r)

    out = pl.pallas_call(
        _pass_b,
        grid=(gi, gj),
        in_specs=[
            pl.BlockSpec((BI, BJ), lambda i, j: (i, j)),
            pl.BlockSpec((BJ, 256), lambda i, j: (j, 0)),
            pl.BlockSpec((BI, 256), lambda i, j: (i, 0)),
            pl.BlockSpec((BI, E), lambda i, j: (i, 0)),
            pl.BlockSpec((BI, E), lambda i, j: (i, 0)),
            pl.BlockSpec((256, 256), lambda i, j: (0, 0)),
            pl.BlockSpec((E, 256), lambda i, j: (0, 0)),
            pl.BlockSpec((1, 256), lambda i, j: (0, 0)),
            pl.BlockSpec((256, 64), lambda i, j: (0, 0)),
            pl.BlockSpec((E, 64), lambda i, j: (0, 0)),
            pl.BlockSpec((1, 64), lambda i, j: (0, 0)),
        ],
        out_specs=pl.BlockSpec((BI, 64), lambda i, j: (i, 0)),
        out_shape=jax.ShapeDtypeStruct((N, 64), f32),
        scratch_shapes=[
            pltpu.VMEM((BI, 256), f32),
        ],
    )(adj, x1, x1, a1, a2, Wn1, We1, bn1r, Wr, Wre, brr)
    return out


# speedup vs baseline: 1.7560x; 1.7560x over previous
import jax
import jax.numpy as jnp
from jax.experimental import pallas as pl
from jax.experimental.pallas import tpu as pltpu

N = 2048
E = 4
BI = 256
BIA = 256
BJ = 1024
LN = 128


def _pass_h0(adj_ref, x0_ref, h0_ref):
    h0_ref[...] = jnp.dot(adj_ref[...], x0_ref[...],
                          preferred_element_type=jnp.float32)


SR = 8


def _pass_a(adj_ref, e_ref, h0_ref, x0i_ref, Wee0_ref, be0_ref, Wee1_ref,
            be1_ref, T_ref, Wn0_ref, We0_ref, bn0_ref,
            x1_ref, a1_ref, a2_ref, acc0, acc1, acc2):
    j = pl.program_id(1)
    nj = pl.num_programs(1)

    w0 = [[Wee0_ref[d, dp] for dp in range(E)] for d in range(E)]
    w1 = [[Wee1_ref[d, dp] for dp in range(E)] for d in range(E)]
    b0 = [be0_ref[d] for d in range(E)]
    b1 = [be1_ref[d] for d in range(E)]

    @pl.when(j == 0)
    def _init():
        acc0[...] = jnp.zeros_like(acc0)
        acc1[...] = jnp.zeros_like(acc1)
        acc2[...] = jnp.zeros_like(acc2)

    for r in range(0, BIA, SR):
        a = adj_ref[r:r + SR, :]
        p = [e_ref[d, r:r + SR, :] for d in range(E)]
        e1 = [jnp.maximum((p[0] * w0[0][dp] + p[1] * w0[1][dp])
                          + (p[2] * w0[2][dp] + p[3] * w0[3][dp]) + b0[dp],
                          0.0)
              for dp in range(E)]
        e2 = [jnp.maximum((e1[0] * w1[0][dp] + e1[1] * w1[1][dp])
                          + (e1[2] * w1[2][dp] + e1[3] * w1[3][dp]) + b1[dp],
                          0.0)
              for dp in range(E)]
        for acc, planes in ((acc0, p), (acc1, e1), (acc2, e2)):
            for d in range(E):
                q = a * planes[d]
                parts = [q[:, c * LN:(c + 1) * LN] for c in range(BJ // LN)]
                while len(parts) > 1:
                    parts = [parts[k] + parts[k + 1]
                             for k in range(0, len(parts), 2)]
                acc[d, r:r + SR, :] += parts[0]

    @pl.when(j == nj - 1)
    def _fin():
        T = T_ref[...]

        def fold(acc):
            s = jnp.concatenate([acc[d] for d in range(E)], axis=1)
            return jnp.dot(s, T, preferred_element_type=jnp.float32)

        ea0 = fold(acc0)
        h = x0i_ref[...] + h0_ref[...]
        x1 = jnp.dot(h, Wn0_ref[...], preferred_element_type=jnp.float32)
        x1 = x1 + jnp.dot(ea0, We0_ref[...], preferred_element_type=jnp.float32)
        x1_ref[...] = jnp.maximum(x1 + bn0_ref[...], 0.0)
        a1_ref[...] = fold(acc1)
        a2_ref[...] = fold(acc2)


def _pass_b(adj_ref, x1j_ref, x1i_ref, a1_ref, a2_ref, Wn1_ref, We1_ref,
            bn1_ref, Wr_ref, Wre_ref, br_ref, out_ref, h1acc):
    j = pl.program_id(1)
    nj = pl.num_programs(1)

    @pl.when(j == 0)
    def _init():
        h1acc[...] = jnp.zeros_like(h1acc)

    h1acc[...] += jnp.dot(adj_ref[...], x1j_ref[...],
                          preferred_element_type=jnp.float32)

    @pl.when(j == nj - 1)
    def _fin():
        h = x1i_ref[...] + h1acc[...]
        x2 = jnp.dot(h, Wn1_ref[...], preferred_element_type=jnp.float32)
        x2 = x2 + jnp.dot(a1_ref[...], We1_ref[...],
                          preferred_element_type=jnp.float32)
        x2 = jnp.maximum(x2 + bn1_ref[...], 0.0)
        out = jnp.dot(x2, Wr_ref[...], preferred_element_type=jnp.float32)
        out = out + jnp.dot(a2_ref[...], Wre_ref[...],
                            preferred_element_type=jnp.float32)
        out_ref[...] = out + br_ref[...]


def kernel(node_features, edge_features, adj, Wn0, We0, bn0, Wee0, be0,
           Wn1, We1, bn1, Wee1, be1, Wr, Wre, br):
    f32 = jnp.float32
    eP = jnp.transpose(edge_features, (2, 0, 1))

    T = jnp.kron(jnp.eye(E, dtype=f32), jnp.ones((LN, 1), dtype=f32))
    bn0r = bn0.reshape(1, -1)
    bn1r = bn1.reshape(1, -1)
    brr = br.reshape(1, -1)

    gi = N // BI
    h0 = pl.pallas_call(
        _pass_h0,
        grid=(gi,),
        in_specs=[
            pl.BlockSpec((BI, N), lambda i: (i, 0)),
            pl.BlockSpec((N, 128), lambda i: (0, 0)),
        ],
        out_specs=pl.BlockSpec((BI, 128), lambda i: (i, 0)),
        out_shape=jax.ShapeDtypeStruct((N, 128), f32),
    )(adj, node_features)

    gj = N // BJ
    gia = N // BIA
    x1, a1, a2 = pl.pallas_call(
        _pass_a,
        grid=(gia, gj),
        in_specs=[
            pl.BlockSpec((BIA, BJ), lambda i, j: (i, j)),
            pl.BlockSpec((E, BIA, BJ), lambda i, j: (0, i, j)),
            pl.BlockSpec((BIA, 128), lambda i, j: (i, 0)),
            pl.BlockSpec((BIA, 128), lambda i, j: (i, 0)),
            pl.BlockSpec(memory_space=pltpu.SMEM),
            pl.BlockSpec(memory_space=pltpu.SMEM),
            pl.BlockSpec(memory_space=pltpu.SMEM),
            pl.BlockSpec(memory_space=pltpu.SMEM),
            pl.BlockSpec((E * LN, E), lambda i, j: (0, 0)),
            pl.BlockSpec((128, 256), lambda i, j: (0, 0)),
            pl.BlockSpec((E, 256), lambda i, j: (0, 0)),
            pl.BlockSpec((1, 256), lambda i, j: (0, 0)),
        ],
        out_specs=[
            pl.BlockSpec((BIA, 256), lambda i, j: (i, 0)),
            pl.BlockSpec((BIA, E), lambda i, j: (i, 0)),
            pl.BlockSpec((BIA, E), lambda i, j: (i, 0)),
        ],
        out_shape=[
            jax.ShapeDtypeStruct((N, 256), f32),
            jax.ShapeDtypeStruct((N, E), f32),
            jax.ShapeDtypeStruct((N, E), f32),
        ],
        scratch_shapes=[
            pltpu.VMEM((E, BIA, LN), f32),
            pltpu.VMEM((E, BIA, LN), f32),
            pltpu.VMEM((E, BIA, LN), f32),
        ],
    )(adj, eP, h0, node_features, Wee0, be0, Wee1, be1, T, Wn0, We0, bn0r)

    out = pl.pallas_call(
        _pass_b,
        grid=(gi, gj),
        in_specs=[
            pl.BlockSpec((BI, BJ), lambda i, j: (i, j)),
            pl.BlockSpec((BJ, 256), lambda i, j: (j, 0)),
            pl.BlockSpec((BI, 256), lambda i, j: (i, 0)),
            pl.BlockSpec((BI, E), lambda i, j: (i, 0)),
            pl.BlockSpec((BI, E), lambda i, j: (i, 0)),
            pl.BlockSpec((256, 256), lambda i, j: (0, 0)),
            pl.BlockSpec((E, 256), lambda i, j: (0, 0)),
            pl.BlockSpec((1, 256), lambda i, j: (0, 0)),
            pl.BlockSpec((256, 64), lambda i, j: (0, 0)),
            pl.BlockSpec((E, 64), lambda i, j: (0, 0)),
            pl.BlockSpec((1, 64), lambda i, j: (0, 0)),
        ],
        out_specs=pl.BlockSpec((BI, 64), lambda i, j: (i, 0)),
        out_shape=jax.ShapeDtypeStruct((N, 64), f32),
        scratch_shapes=[
            pltpu.VMEM((BI, 256), f32),
        ],
    )(adj, x1, x1, a1, a2, Wn1, We1, bn1r, Wr, Wre, brr)
    return out


# speedup vs baseline: 1.8197x; 1.0363x over previous
import jax
import jax.numpy as jnp
from jax.experimental import pallas as pl
from jax.experimental.pallas import tpu as pltpu

N = 2048
E = 4
BI = 256
BIA = 256
BJ = 2048
LN = 128


def _pass_h0(adj_ref, x0_ref, h0_ref):
    h0_ref[...] = jnp.dot(adj_ref[...], x0_ref[...],
                          preferred_element_type=jnp.float32)


SR = 8


def _pass_a(adj_ref, e_ref, h0_ref, x0i_ref, Wee0_ref, be0_ref, Wee1_ref,
            be1_ref, T_ref, Wn0_ref, We0_ref, bn0_ref,
            x1_ref, a1_ref, a2_ref, acc0, acc1, acc2):
    j = pl.program_id(1)
    nj = pl.num_programs(1)

    w0 = [[Wee0_ref[d, dp] for dp in range(E)] for d in range(E)]
    w1 = [[Wee1_ref[d, dp] for dp in range(E)] for d in range(E)]
    b0 = [be0_ref[d] for d in range(E)]
    b1 = [be1_ref[d] for d in range(E)]

    @pl.when(j == 0)
    def _init():
        acc0[...] = jnp.zeros_like(acc0)
        acc1[...] = jnp.zeros_like(acc1)
        acc2[...] = jnp.zeros_like(acc2)

    for r in range(0, BIA, SR):
        a = adj_ref[r:r + SR, :]
        p = [e_ref[d, r:r + SR, :] for d in range(E)]
        e1 = [jnp.maximum((p[0] * w0[0][dp] + p[1] * w0[1][dp])
                          + (p[2] * w0[2][dp] + p[3] * w0[3][dp]) + b0[dp],
                          0.0)
              for dp in range(E)]
        e2 = [jnp.maximum((e1[0] * w1[0][dp] + e1[1] * w1[1][dp])
                          + (e1[2] * w1[2][dp] + e1[3] * w1[3][dp]) + b1[dp],
                          0.0)
              for dp in range(E)]
        for acc, planes in ((acc0, p), (acc1, e1), (acc2, e2)):
            for d in range(E):
                q = a * planes[d]
                parts = [q[:, c * LN:(c + 1) * LN] for c in range(BJ // LN)]
                while len(parts) > 1:
                    parts = [parts[k] + parts[k + 1]
                             for k in range(0, len(parts), 2)]
                acc[d, r:r + SR, :] += parts[0]

    @pl.when(j == nj - 1)
    def _fin():
        T = T_ref[...]

        def fold(acc):
            s = jnp.concatenate([acc[d] for d in range(E)], axis=1)
            return jnp.dot(s, T, preferred_element_type=jnp.float32)

        ea0 = fold(acc0)
        h = x0i_ref[...] + h0_ref[...]
        x1 = jnp.dot(h, Wn0_ref[...], preferred_element_type=jnp.float32)
        x1 = x1 + jnp.dot(ea0, We0_ref[...], preferred_element_type=jnp.float32)
        x1_ref[...] = jnp.maximum(x1 + bn0_ref[...], 0.0)
        a1_ref[...] = fold(acc1)
        a2_ref[...] = fold(acc2)


def _pass_b(adj_ref, x1j_ref, x1i_ref, a1_ref, a2_ref, Wn1_ref, We1_ref,
            bn1_ref, Wr_ref, Wre_ref, br_ref, out_ref, h1acc):
    j = pl.program_id(1)
    nj = pl.num_programs(1)

    @pl.when(j == 0)
    def _init():
        h1acc[...] = jnp.zeros_like(h1acc)

    h1acc[...] += jnp.dot(adj_ref[...], x1j_ref[...],
                          preferred_element_type=jnp.float32)

    @pl.when(j == nj - 1)
    def _fin():
        h = x1i_ref[...] + h1acc[...]
        x2 = jnp.dot(h, Wn1_ref[...], preferred_element_type=jnp.float32)
        x2 = x2 + jnp.dot(a1_ref[...], We1_ref[...],
                          preferred_element_type=jnp.float32)
        x2 = jnp.maximum(x2 + bn1_ref[...], 0.0)
        out = jnp.dot(x2, Wr_ref[...], preferred_element_type=jnp.float32)
        out = out + jnp.dot(a2_ref[...], Wre_ref[...],
                            preferred_element_type=jnp.float32)
        out_ref[...] = out + br_ref[...]


def kernel(node_features, edge_features, adj, Wn0, We0, bn0, Wee0, be0,
           Wn1, We1, bn1, Wee1, be1, Wr, Wre, br):
    f32 = jnp.float32
    eP = jnp.transpose(edge_features, (2, 0, 1))

    T = jnp.kron(jnp.eye(E, dtype=f32), jnp.ones((LN, 1), dtype=f32))
    bn0r = bn0.reshape(1, -1)
    bn1r = bn1.reshape(1, -1)
    brr = br.reshape(1, -1)

    gi = N // BI
    h0 = pl.pallas_call(
        _pass_h0,
        grid=(gi,),
        in_specs=[
            pl.BlockSpec((BI, N), lambda i: (i, 0)),
            pl.BlockSpec((N, 128), lambda i: (0, 0)),
        ],
        out_specs=pl.BlockSpec((BI, 128), lambda i: (i, 0)),
        out_shape=jax.ShapeDtypeStruct((N, 128), f32),
    )(adj, node_features)

    gj = N // BJ
    gia = N // BIA
    x1, a1, a2 = pl.pallas_call(
        _pass_a,
        grid=(gia, gj),
        in_specs=[
            pl.BlockSpec((BIA, BJ), lambda i, j: (i, j)),
            pl.BlockSpec((E, BIA, BJ), lambda i, j: (0, i, j)),
            pl.BlockSpec((BIA, 128), lambda i, j: (i, 0)),
            pl.BlockSpec((BIA, 128), lambda i, j: (i, 0)),
            pl.BlockSpec(memory_space=pltpu.SMEM),
            pl.BlockSpec(memory_space=pltpu.SMEM),
            pl.BlockSpec(memory_space=pltpu.SMEM),
            pl.BlockSpec(memory_space=pltpu.SMEM),
            pl.BlockSpec((E * LN, E), lambda i, j: (0, 0)),
            pl.BlockSpec((128, 256), lambda i, j: (0, 0)),
            pl.BlockSpec((E, 256), lambda i, j: (0, 0)),
            pl.BlockSpec((1, 256), lambda i, j: (0, 0)),
        ],
        out_specs=[
            pl.BlockSpec((BIA, 256), lambda i, j: (i, 0)),
            pl.BlockSpec((BIA, E), lambda i, j: (i, 0)),
            pl.BlockSpec((BIA, E), lambda i, j: (i, 0)),
        ],
        out_shape=[
            jax.ShapeDtypeStruct((N, 256), f32),
            jax.ShapeDtypeStruct((N, E), f32),
            jax.ShapeDtypeStruct((N, E), f32),
        ],
        scratch_shapes=[
            pltpu.VMEM((E, BIA, LN), f32),
            pltpu.VMEM((E, BIA, LN), f32),
            pltpu.VMEM((E, BIA, LN), f32),
        ],
    )(adj, eP, h0, node_features, Wee0, be0, Wee1, be1, T, Wn0, We0, bn0r)

    out = pl.pallas_call(
        _pass_b,
        grid=(gi, gj),
        in_specs=[
            pl.BlockSpec((BI, BJ), lambda i, j: (i, j)),
            pl.BlockSpec((BJ, 256), lambda i, j: (j, 0)),
            pl.BlockSpec((BI, 256), lambda i, j: (i, 0)),
            pl.BlockSpec((BI, E), lambda i, j: (i, 0)),
            pl.BlockSpec((BI, E), lambda i, j: (i, 0)),
            pl.BlockSpec((256, 256), lambda i, j: (0, 0)),
            pl.BlockSpec((E, 256), lambda i, j: (0, 0)),
            pl.BlockSpec((1, 256), lambda i, j: (0, 0)),
            pl.BlockSpec((256, 64), lambda i, j: (0, 0)),
            pl.BlockSpec((E, 64), lambda i, j: (0, 0)),
            pl.BlockSpec((1, 64), lambda i, j: (0, 0)),
        ],
        out_specs=pl.BlockSpec((BI, 64), lambda i, j: (i, 0)),
        out_shape=jax.ShapeDtypeStruct((N, 64), f32),
        scratch_shapes=[
            pltpu.VMEM((BI, 256), f32),
        ],
    )(adj, x1, x1, a1, a2, Wn1, We1, bn1r, Wr, Wre, brr)
    return out
